# baseline (speedup 1.0000x reference)
import functools

import jax
import jax.numpy as jnp
import numpy as np
from jax import lax
from jax.experimental import pallas as pl
from jax.experimental.pallas import tpu as pltpu
from jax.experimental.pallas import tpu_sc as plsc

L = 128
N = L * L
NUM_CORES = 2
NUM_SUBCORES = 16
NUM_WORKERS = NUM_CORES * NUM_SUBCORES
LANES = 16


def _perm_flat():
    p = np.arange(N, dtype=np.int32).reshape(L, L)
    for i in range(L):
        p[:, i] = np.roll(p[:, i], shift=-i)
    return jnp.asarray(p.reshape(-1))


def _make_sc_permute(batch):
    assert batch % NUM_WORKERS == 0
    rows_per_worker = batch // NUM_WORKERS
    mesh = plsc.VectorSubcoreMesh(
        core_axis_name="c",
        subcore_axis_name="s",
        num_cores=NUM_CORES,
        num_subcores=NUM_SUBCORES,
    )

    @functools.partial(
        pl.kernel,
        out_type=jax.ShapeDtypeStruct((batch, N), jnp.float32),
        mesh=mesh,
        scratch_types=[
            pltpu.VMEM((N,), jnp.int32),
            pltpu.VMEM((N,), jnp.float32),
            pltpu.VMEM((N,), jnp.float32),
        ],
        compiler_params=pltpu.CompilerParams(needs_layout_passes=False),
    )
    def permute(x_hbm, perm_hbm, out_hbm, idx_v, xin_v, out_v):
        wid = lax.axis_index("s") * NUM_CORES + lax.axis_index("c")
        base = wid * rows_per_worker
        pltpu.sync_copy(perm_hbm, idx_v)

        def row_body(i, carry):
            b = base + i
            pltpu.sync_copy(x_hbm.at[b], xin_v)

            def gather_body(j, carry2):
                sl = pl.ds(j * LANES, LANES)
                out_v[sl] = plsc.load_gather(xin_v, [idx_v[sl]])
                return carry2

            lax.fori_loop(0, N // LANES, gather_body, 0)
            pltpu.sync_copy(out_v, out_hbm.at[b])
            return carry

        lax.fori_loop(0, rows_per_worker, row_body, 0)

    return permute


def kernel(x):
    batch = x.shape[0]
    x = x.reshape(batch, N)
    out = _make_sc_permute(batch)(x, _perm_flat())
    return out.reshape(batch, 1, L, L)


# speedup vs baseline: 1.7357x; 1.7357x over previous
import functools

import jax
import jax.numpy as jnp
import numpy as np
from jax import lax
from jax.experimental import pallas as pl
from jax.experimental.pallas import tpu as pltpu
from jax.experimental.pallas import tpu_sc as plsc

L = 128
N = L * L
NUM_CORES = 2
NUM_SUBCORES = 16
NUM_WORKERS = NUM_CORES * NUM_SUBCORES
LANES = 16
UNROLL = 8


def _perm_flat():
    p = np.arange(N, dtype=np.int32).reshape(L, L)
    for i in range(L):
        p[:, i] = np.roll(p[:, i], shift=-i)
    return jnp.asarray(p.reshape(-1))


def _make_sc_permute(batch):
    assert batch % (2 * NUM_WORKERS) == 0
    rows_per_worker = batch // NUM_WORKERS
    pairs = rows_per_worker // 2
    mesh = plsc.VectorSubcoreMesh(
        core_axis_name="c",
        subcore_axis_name="s",
        num_cores=NUM_CORES,
        num_subcores=NUM_SUBCORES,
    )

    @functools.partial(
        pl.kernel,
        out_type=jax.ShapeDtypeStruct((batch, N), jnp.float32),
        mesh=mesh,
        scratch_types=[
            pltpu.VMEM((N,), jnp.int32),
            pltpu.VMEM((N,), jnp.float32),
            pltpu.VMEM((N,), jnp.float32),
            pltpu.VMEM((N,), jnp.float32),
            pltpu.VMEM((N,), jnp.float32),
            pltpu.SemaphoreType.DMA,
            pltpu.SemaphoreType.DMA,
            pltpu.SemaphoreType.DMA,
            pltpu.SemaphoreType.DMA,
        ],
        compiler_params=pltpu.CompilerParams(needs_layout_passes=False),
    )
    def permute(x_hbm, perm_hbm, out_hbm, idx_v, xin0_v, xin1_v,
                out0_v, out1_v, in_sem0, in_sem1, out_sem0, out_sem1):
        wid = lax.axis_index("s") * NUM_CORES + lax.axis_index("c")
        base = wid * rows_per_worker
        xin_bufs = (xin0_v, xin1_v)
        out_bufs = (out0_v, out1_v)
        in_sems = (in_sem0, in_sem1)
        out_sems = (out_sem0, out_sem1)

        pltpu.sync_copy(perm_hbm, idx_v)
        pltpu.async_copy(x_hbm.at[base], xin0_v, in_sem0)
        pltpu.async_copy(x_hbm.at[base + 1], xin1_v, in_sem1)

        def pair_body(i, carry):
            for b in range(2):
                row = base + 2 * i + b
                pltpu.make_async_copy(
                    x_hbm.at[row], xin_bufs[b], in_sems[b]).wait()
                @pl.when(i > 0)
                def _wait_out():
                    pltpu.make_async_copy(
                        out_bufs[b], out_hbm.at[row - 2], out_sems[b]).wait()

                def gather_body(j, carry2):
                    for u in range(UNROLL):
                        sl = pl.ds(j * (LANES * UNROLL) + u * LANES, LANES)
                        out_bufs[b][sl] = plsc.load_gather(
                            xin_bufs[b], [idx_v[sl]])
                    return carry2

                lax.fori_loop(0, N // (LANES * UNROLL), gather_body, 0,
                              unroll=False)
                pltpu.async_copy(out_bufs[b], out_hbm.at[row], out_sems[b])

                @pl.when(i < pairs - 1)
                def _refill():
                    pltpu.async_copy(
                        x_hbm.at[row + 2], xin_bufs[b], in_sems[b])
            return carry

        lax.fori_loop(0, pairs, pair_body, 0, unroll=False)
        last = base + 2 * (pairs - 1)
        pltpu.make_async_copy(out0_v, out_hbm.at[last], out_sem0).wait()
        pltpu.make_async_copy(out1_v, out_hbm.at[last + 1], out_sem1).wait()

    return permute


def kernel(x):
    batch = x.shape[0]
    x = x.reshape(batch, N)
    out = _make_sc_permute(batch)(x, _perm_flat())
    return out.reshape(batch, 1, L, L)


# speedup vs baseline: 3.1841x; 1.8344x over previous
import functools

import jax
import jax.numpy as jnp
import numpy as np
from jax import lax
from jax.experimental import pallas as pl
from jax.experimental.pallas import tpu as pltpu
from jax.experimental.pallas import tpu_sc as plsc

L = 128
N = L * L
NUM_CORES = 2
NUM_SUBCORES = 16
NUM_WORKERS = NUM_CORES * NUM_SUBCORES
LANES = 16
UNROLL = 8


def _perm_flat():
    p = np.arange(N, dtype=np.int32).reshape(L, L)
    for i in range(L):
        p[:, i] = np.roll(p[:, i], shift=-i)
    return jnp.asarray(p.reshape(-1))


def _make_sc_permute(batch):
    assert batch % (2 * NUM_WORKERS) == 0
    rows_per_worker = batch // NUM_WORKERS
    pairs = rows_per_worker // 2
    mesh = plsc.VectorSubcoreMesh(
        core_axis_name="c",
        subcore_axis_name="s",
        num_cores=NUM_CORES,
        num_subcores=NUM_SUBCORES,
    )

    @functools.partial(
        pl.kernel,
        out_type=jax.ShapeDtypeStruct((batch, N), jnp.float32),
        mesh=mesh,
        scratch_types=[
            pltpu.VMEM((N,), jnp.int32),
            pltpu.VMEM((N,), jnp.float32),
            pltpu.VMEM((N,), jnp.float32),
            pltpu.VMEM((N,), jnp.float32),
            pltpu.VMEM((N,), jnp.float32),
            pltpu.SemaphoreType.DMA,
            pltpu.SemaphoreType.DMA,
            pltpu.SemaphoreType.DMA,
            pltpu.SemaphoreType.DMA,
        ],
        compiler_params=pltpu.CompilerParams(needs_layout_passes=False),
    )
    def permute(x_hbm, perm_hbm, out_hbm, idx_v, xin0_v, xin1_v,
                out0_v, out1_v, in_sem0, in_sem1, out_sem0, out_sem1):
        wid = lax.axis_index("s") * NUM_CORES + lax.axis_index("c")
        base = wid * rows_per_worker
        xin_bufs = (xin0_v, xin1_v)
        out_bufs = (out0_v, out1_v)
        in_sems = (in_sem0, in_sem1)
        out_sems = (out_sem0, out_sem1)

        pltpu.sync_copy(perm_hbm, idx_v)
        pltpu.async_copy(x_hbm.at[base], xin0_v, in_sem0)
        pltpu.async_copy(x_hbm.at[base + 1], xin1_v, in_sem1)

        def pair_body(i, carry):
            for b in range(2):
                row = base + 2 * i + b
                pltpu.make_async_copy(
                    x_hbm.at[row], xin_bufs[b], in_sems[b]).wait()
                @pl.when(i > 0)
                def _wait_out():
                    pltpu.make_async_copy(
                        out_bufs[b], out_hbm.at[row - 2], out_sems[b]).wait()

                @plsc.parallel_loop(0, N // LANES, step=1, unroll=UNROLL)
                def _gather(j):
                    sl = pl.ds(j * LANES, LANES)
                    out_bufs[b][sl] = plsc.load_gather(
                        xin_bufs[b], [idx_v[sl]])
                pltpu.async_copy(out_bufs[b], out_hbm.at[row], out_sems[b])

                @pl.when(i < pairs - 1)
                def _refill():
                    pltpu.async_copy(
                        x_hbm.at[row + 2], xin_bufs[b], in_sems[b])
            return carry

        lax.fori_loop(0, pairs, pair_body, 0, unroll=False)
        last = base + 2 * (pairs - 1)
        pltpu.make_async_copy(out0_v, out_hbm.at[last], out_sem0).wait()
        pltpu.make_async_copy(out1_v, out_hbm.at[last + 1], out_sem1).wait()

    return permute


def kernel(x):
    batch = x.shape[0]
    x = x.reshape(batch, N)
    out = _make_sc_permute(batch)(x, _perm_flat())
    return out.reshape(batch, 1, L, L)


# speedup vs baseline: 3.1951x; 1.0035x over previous
import functools

import jax
import jax.numpy as jnp
import numpy as np
from jax import lax
from jax.experimental import pallas as pl
from jax.experimental.pallas import tpu as pltpu
from jax.experimental.pallas import tpu_sc as plsc

L = 128
N = L * L
NUM_CORES = 2
NUM_SUBCORES = 16
NUM_WORKERS = NUM_CORES * NUM_SUBCORES
LANES = 16
UNROLL = 16


def _perm_flat():
    p = np.arange(N, dtype=np.int32).reshape(L, L)
    for i in range(L):
        p[:, i] = np.roll(p[:, i], shift=-i)
    return jnp.asarray(p.reshape(-1))


def _make_sc_permute(batch):
    assert batch % (2 * NUM_WORKERS) == 0
    rows_per_worker = batch // NUM_WORKERS
    pairs = rows_per_worker // 2
    mesh = plsc.VectorSubcoreMesh(
        core_axis_name="c",
        subcore_axis_name="s",
        num_cores=NUM_CORES,
        num_subcores=NUM_SUBCORES,
    )

    @functools.partial(
        pl.kernel,
        out_type=jax.ShapeDtypeStruct((batch, N), jnp.float32),
        mesh=mesh,
        scratch_types=[
            pltpu.VMEM((N,), jnp.int32),
            pltpu.VMEM((N,), jnp.float32),
            pltpu.VMEM((N,), jnp.float32),
            pltpu.VMEM((N,), jnp.float32),
            pltpu.VMEM((N,), jnp.float32),
            pltpu.SemaphoreType.DMA,
            pltpu.SemaphoreType.DMA,
            pltpu.SemaphoreType.DMA,
            pltpu.SemaphoreType.DMA,
        ],
        compiler_params=pltpu.CompilerParams(needs_layout_passes=False),
    )
    def permute(x_hbm, perm_hbm, out_hbm, idx_v, xin0_v, xin1_v,
                out0_v, out1_v, in_sem0, in_sem1, out_sem0, out_sem1):
        wid = lax.axis_index("s") * NUM_CORES + lax.axis_index("c")
        base = wid * rows_per_worker
        xin_bufs = (xin0_v, xin1_v)
        out_bufs = (out0_v, out1_v)
        in_sems = (in_sem0, in_sem1)
        out_sems = (out_sem0, out_sem1)

        pltpu.sync_copy(perm_hbm, idx_v)
        pltpu.async_copy(x_hbm.at[base], xin0_v, in_sem0)
        pltpu.async_copy(x_hbm.at[base + 1], xin1_v, in_sem1)

        def pair_body(i, carry):
            for b in range(2):
                row = base + 2 * i + b
                pltpu.make_async_copy(
                    x_hbm.at[row], xin_bufs[b], in_sems[b]).wait()
                @pl.when(i > 0)
                def _wait_out():
                    pltpu.make_async_copy(
                        out_bufs[b], out_hbm.at[row - 2], out_sems[b]).wait()

                @plsc.parallel_loop(0, N // LANES, step=1, unroll=UNROLL)
                def _gather(j):
                    sl = pl.ds(j * LANES, LANES)
                    out_bufs[b][sl] = plsc.load_gather(
                        xin_bufs[b], [idx_v[sl]])
                pltpu.async_copy(out_bufs[b], out_hbm.at[row], out_sems[b])

                @pl.when(i < pairs - 1)
                def _refill():
                    pltpu.async_copy(
                        x_hbm.at[row + 2], xin_bufs[b], in_sems[b])
            return carry

        lax.fori_loop(0, pairs, pair_body, 0, unroll=False)
        last = base + 2 * (pairs - 1)
        pltpu.make_async_copy(out0_v, out_hbm.at[last], out_sem0).wait()
        pltpu.make_async_copy(out1_v, out_hbm.at[last + 1], out_sem1).wait()

    return permute


def kernel(x):
    batch = x.shape[0]
    x = x.reshape(batch, N)
    out = _make_sc_permute(batch)(x, _perm_flat())
    return out.reshape(batch, 1, L, L)


# speedup vs baseline: 5.8126x; 1.8192x over previous
import functools

import jax
import jax.numpy as jnp
import numpy as np
from jax import lax
from jax.experimental import pallas as pl
from jax.experimental.pallas import tpu as pltpu
from jax.experimental.pallas import tpu_sc as plsc

L = 128
N = L * L
NUM_CORES = 2
NUM_SUBCORES = 16
NUM_WORKERS = NUM_CORES * NUM_SUBCORES
LANES = 16
CHUNKS = L // LANES
UNROLL = 2


def _perm_flat():
    p = np.arange(N, dtype=np.int32).reshape(L, L)
    for i in range(L):
        p[:, i] = np.roll(p[:, i], shift=-i)
    return jnp.asarray(p.reshape(-1))


def _make_sc_permute(batch):
    assert batch % (2 * NUM_WORKERS) == 0
    rows_per_worker = batch // NUM_WORKERS
    pairs = rows_per_worker // 2
    mesh = plsc.VectorSubcoreMesh(
        core_axis_name="c",
        subcore_axis_name="s",
        num_cores=NUM_CORES,
        num_subcores=NUM_SUBCORES,
    )

    @functools.partial(
        pl.kernel,
        out_type=jax.ShapeDtypeStruct((batch, 1, L, L), jnp.float32),
        mesh=mesh,
        scratch_types=[
            pltpu.VMEM((N,), jnp.int32),
            pltpu.VMEM((N,), jnp.float32),
            pltpu.VMEM((N,), jnp.float32),
            pltpu.VMEM((1, L, L), jnp.float32),
            pltpu.VMEM((1, L, L), jnp.float32),
            pltpu.SemaphoreType.DMA,
            pltpu.SemaphoreType.DMA,
            pltpu.SemaphoreType.DMA,
            pltpu.SemaphoreType.DMA,
        ],
        compiler_params=pltpu.CompilerParams(needs_layout_passes=False),
    )
    def permute(x_hbm, perm_hbm, out_hbm, idx_v, xin0_v, xin1_v,
                out0_v, out1_v, in_sem0, in_sem1, out_sem0, out_sem1):
        wid = lax.axis_index("s") * NUM_CORES + lax.axis_index("c")
        base = wid * rows_per_worker
        xin_bufs = (xin0_v, xin1_v)
        out_bufs = (out0_v, out1_v)
        in_sems = (in_sem0, in_sem1)
        out_sems = (out_sem0, out_sem1)

        pltpu.sync_copy(perm_hbm, idx_v)
        pltpu.async_copy(x_hbm.at[base], xin0_v, in_sem0)
        pltpu.async_copy(x_hbm.at[base + 1], xin1_v, in_sem1)

        def pair_body(i, carry):
            for b in range(2):
                row = base + 2 * i + b
                pltpu.make_async_copy(
                    x_hbm.at[row], xin_bufs[b], in_sems[b]).wait()
                @pl.when(i > 0)
                def _wait_out():
                    pltpu.make_async_copy(
                        out_bufs[b], out_hbm.at[row - 2], out_sems[b]).wait()

                @plsc.parallel_loop(0, L, step=1, unroll=UNROLL)
                def _gather(r):
                    for u in range(CHUNKS):
                        sl = pl.ds(r * L + u * LANES, LANES)
                        out_bufs[b][0, r, pl.ds(u * LANES, LANES)] = (
                            plsc.load_gather(xin_bufs[b], [idx_v[sl]]))
                pltpu.async_copy(out_bufs[b], out_hbm.at[row], out_sems[b])

                @pl.when(i < pairs - 1)
                def _refill():
                    pltpu.async_copy(
                        x_hbm.at[row + 2], xin_bufs[b], in_sems[b])
            return carry

        lax.fori_loop(0, pairs, pair_body, 0, unroll=False)
        last = base + 2 * (pairs - 1)
        pltpu.make_async_copy(out0_v, out_hbm.at[last], out_sem0).wait()
        pltpu.make_async_copy(out1_v, out_hbm.at[last + 1], out_sem1).wait()

    return permute


def kernel(x):
    batch = x.shape[0]
    x = x.reshape(batch, N)
    return _make_sc_permute(batch)(x, _perm_flat())
